# baseline (speedup 1.0000x reference)
import functools

import jax
import jax.numpy as jnp
from jax import lax
from jax.experimental import pallas as pl
from jax.experimental.pallas import tpu as pltpu
from jax.experimental.pallas import tpu_sc as plsc

_LB = 8192
_G = 8
_LG = _LB // _G


_CH = 1024
_NCH = 976
_T = _NCH * _CH


def _repack_sc(table_t):
    D = table_t.shape[0]
    Q = _T // 8
    info = plsc.get_sparse_core_info()
    nc, ns = info.num_cores, info.num_subcores
    nw = nc * ns
    per_w = _NCH // nw
    mesh = plsc.VectorSubcoreMesh(core_axis_name="c", subcore_axis_name="s")

    @functools.partial(
        pl.kernel,
        mesh=mesh,
        compiler_params=pltpu.CompilerParams(needs_layout_passes=False),
        out_type=jax.ShapeDtypeStruct((Q, 128), jnp.int32),
        scratch_types=[
            pltpu.VMEM((D, _CH), jnp.float32),
            pltpu.VMEM((_CH // 8, 128), jnp.int32),
            pltpu.SemaphoreType.DMA,
        ],
    )
    def rk(table_hbm, out_hbm, buf, stage, sem):
        wid = lax.axis_index("s") * nc + lax.axis_index("c")
        iota = lax.iota(jnp.int32, 16)

        @pl.loop(0, per_w + 1)
        def _(j):
            c = wid + j * nw

            @pl.when(c < _NCH)
            def _():
                pltpu.async_copy(
                    table_hbm.at[:, pl.ds(c * _CH, _CH)], buf, sem
                ).wait()

                @pl.loop(0, _CH, step=16)
                def _(pl0):
                    pls = pl0 + iota
                    r_vec = lax.shift_right_logical(pls, 3)
                    cb = (pls & 7) * 16
                    for wp in range(16):
                        lo = plsc.load_gather(
                            buf, [jnp.full((16,), wp, jnp.int32), pls]
                        )
                        hi = plsc.load_gather(
                            buf, [jnp.full((16,), wp + 16, jnp.int32), pls]
                        )
                        ulo = plsc.bitcast(lo, jnp.int32)
                        uhi = plsc.bitcast(hi, jnp.int32)
                        w = (uhi & jnp.int32(-65536)) | lax.shift_right_logical(
                            ulo, 16
                        )
                        plsc.store_scatter(stage, [r_vec, cb + wp], w)

                pltpu.async_copy(
                    stage, out_hbm.at[pl.ds(c * (_CH // 8), _CH // 8)], sem
                ).wait()

    return rk(table_t)


def _gather_sc(table_c, widx3):
    nw = widx3.shape[0]
    B = nw * 512
    info = plsc.get_sparse_core_info()
    nc = info.num_cores
    mesh = plsc.VectorSubcoreMesh(core_axis_name="c", subcore_axis_name="s")

    @functools.partial(
        pl.kernel,
        mesh=mesh,
        out_type=jax.ShapeDtypeStruct((B, 128), jnp.int32),
        scratch_types=[
            pltpu.VMEM((4, 128), jnp.int32),
            pltpu.VMEM((4, 128, 128), jnp.int32),
            pltpu.SemaphoreType.DMA,
        ],
    )
    def gk(widx_hbm, table_hbm, out_hbm, idx_v, rows_v, sem):
        wid = lax.axis_index("s") * nc + lax.axis_index("c")
        base = wid * 512
        pltpu.sync_copy(widx_hbm.at[wid], idx_v)
        copies = [
            pltpu.async_copy(table_hbm.at[idx_v.at[k]], rows_v.at[k], sem)
            for k in range(4)
        ]
        for k in range(4):
            copies[k].wait()
            pltpu.sync_copy(
                rows_v.at[k], out_hbm.at[pl.ds(base + 128 * k, 128)]
            )

    return gk(widx3, table_c)


def _mlp_body(x_ref, o_ref_in, pf_ref, wlo_ref, whi_ref, w1_ref, tt_ref,
              b1_ref, w2t_ref, b2_ref, o_ref):
    xw = x_ref[...]
    grp = jax.lax.broadcasted_iota(jnp.int32, xw.shape, 1) // 16
    xm = jnp.where(grp == o_ref_in[...], xw, 0)
    xlo = lax.bitcast_convert_type(
        lax.shift_left(xm, 16), jnp.float32
    ).astype(jnp.bfloat16)
    xhi = lax.bitcast_convert_type(
        xm & jnp.int32(-65536), jnp.float32
    ).astype(jnp.bfloat16)
    h = lax.dot_general(
        wlo_ref[...], xlo,
        dimension_numbers=(((0,), (1,)), ((), ())),
        preferred_element_type=jnp.float32,
    )
    h = h + lax.dot_general(
        whi_ref[...], xhi,
        dimension_numbers=(((0,), (1,)), ((), ())),
        preferred_element_type=jnp.float32,
    )
    nt = tt_ref.shape[0]
    oh = (
        jax.lax.broadcasted_iota(jnp.int32, (xw.shape[0], nt), 1)
        == pf_ref[...]
    ).astype(jnp.bfloat16)
    t = lax.dot_general(
        oh, tt_ref[...],
        dimension_numbers=(((1,), (0,)), ((), ())),
        preferred_element_type=jnp.float32,
    ).astype(jnp.bfloat16)
    h = h + lax.dot_general(
        w1_ref[...], t,
        dimension_numbers=(((0,), (1,)), ((), ())),
        preferred_element_type=jnp.float32,
    )
    h = jnp.maximum(h + b1_ref[...], 0.0).astype(jnp.bfloat16)
    z = jnp.dot(w2t_ref[...], h, preferred_element_type=jnp.float32)
    z = z + b2_ref[...]
    o_ref[...] = jax.nn.sigmoid(z)


def _mlp_tc_t(x128, o_col, pf_col, Wlo, Whi, W1b, tab_tail, b1c, W2t, b2c):
    B = x128.shape[0]
    H = Wlo.shape[1]
    N = W2t.shape[0]
    D = W1b.shape[0]
    NT = tab_tail.shape[0]
    BB = 2048
    return pl.pallas_call(
        _mlp_body,
        grid=(B // BB,),
        in_specs=[
            pl.BlockSpec((BB, 128), lambda i: (i, 0)),
            pl.BlockSpec((BB, 1), lambda i: (i, 0)),
            pl.BlockSpec((BB, 1), lambda i: (i, 0)),
            pl.BlockSpec((128, H), lambda i: (0, 0)),
            pl.BlockSpec((128, H), lambda i: (0, 0)),
            pl.BlockSpec((D, H), lambda i: (0, 0)),
            pl.BlockSpec((NT, D), lambda i: (0, 0)),
            pl.BlockSpec((H, 1), lambda i: (0, 0)),
            pl.BlockSpec((N, H), lambda i: (0, 0)),
            pl.BlockSpec((N, 1), lambda i: (0, 0)),
        ],
        out_specs=pl.BlockSpec((N, BB), lambda i: (0, i)),
        out_shape=jax.ShapeDtypeStruct((N, B), jnp.float32),
    )(x128, o_col, pf_col, Wlo, Whi, W1b, tab_tail, b1c, W2t, b2c)


def kernel(indices, table, W1, b1, W2, b2):
    table_t = jnp.transpose(table)
    table_c = _repack_sc(table_t)
    p = indices.astype(jnp.int32)
    tail = p >= _T
    q = jnp.where(tail, 0, lax.shift_right_logical(p, 3))
    widx3 = q.reshape(-1, 4, 128)
    o_col = jnp.where(tail, 8, p & 7).reshape(-1, 1)
    pf_col = jnp.where(tail, p - _T, -1).reshape(-1, 1)
    x128 = _gather_sc(table_c, widx3)
    Wlo = jnp.tile(W1[:16], (_G, 1)).astype(jnp.bfloat16)
    Whi = jnp.tile(W1[16:], (_G, 1)).astype(jnp.bfloat16)
    W1b = W1.astype(jnp.bfloat16)
    tab_tail = table[_T:].astype(jnp.bfloat16)
    W2t = jnp.transpose(W2).astype(jnp.bfloat16)
    out_t = _mlp_tc_t(
        x128, o_col, pf_col, Wlo, Whi, W1b, tab_tail,
        b1.reshape(-1, 1), W2t, b2.reshape(-1, 1)
    )
    return jnp.transpose(out_t)


# speedup vs baseline: 1.0588x; 1.0588x over previous
import functools

import jax
import jax.numpy as jnp
from jax import lax
from jax.experimental import pallas as pl
from jax.experimental.pallas import tpu as pltpu
from jax.experimental.pallas import tpu_sc as plsc

_LB = 8192
_G = 8
_LG = _LB // _G


def _round_bf16_bits(u):
    return lax.shift_right_arithmetic(
        u + 0x7FFF + (lax.shift_right_logical(u, 16) & 1), 16
    )


def _relayout_body(x_ref, o_ref):
    x = x_ref[...]
    u = lax.bitcast_convert_type(x, jnp.int32)
    lo = _round_bf16_bits(u[:16, :]) & 0xFFFF
    hi = lax.shift_left(_round_bf16_bits(u[16:, :]), 16)
    w = hi | lo
    parts = [
        jnp.transpose(w[:, _LG * o : _LG * (o + 1)]) for o in range(_G)
    ]
    o_ref[...] = jnp.concatenate(parts, axis=1)


def _relayout_tc(table_t):
    D, V = table_t.shape
    grid = (V + _LB - 1) // _LB
    Q = grid * _LG
    return pl.pallas_call(
        _relayout_body,
        grid=(grid,),
        in_specs=[pl.BlockSpec((D, _LB), lambda i: (0, i))],
        out_specs=pl.BlockSpec((_LG, 128), lambda i: (i, 0)),
        out_shape=jax.ShapeDtypeStruct((Q, 128), jnp.int32),
    )(table_t)


def _gather_sc(table_c, widx3):
    nw = widx3.shape[0]
    B = nw * 512
    info = plsc.get_sparse_core_info()
    nc = info.num_cores
    mesh = plsc.VectorSubcoreMesh(core_axis_name="c", subcore_axis_name="s")

    @functools.partial(
        pl.kernel,
        mesh=mesh,
        out_type=jax.ShapeDtypeStruct((B, 128), jnp.int32),
        scratch_types=[
            pltpu.VMEM((4, 128), jnp.int32),
            pltpu.VMEM((4, 128, 128), jnp.int32),
            pltpu.SemaphoreType.DMA,
        ],
    )
    def gk(widx_hbm, table_hbm, out_hbm, idx_v, rows_v, sem):
        wid = lax.axis_index("s") * nc + lax.axis_index("c")
        base = wid * 512
        pltpu.sync_copy(widx_hbm.at[wid], idx_v)
        copies = [
            pltpu.async_copy(table_hbm.at[idx_v.at[k]], rows_v.at[k], sem)
            for k in range(4)
        ]
        for k in range(4):
            copies[k].wait()
            pltpu.sync_copy(
                rows_v.at[k], out_hbm.at[pl.ds(base + 128 * k, 128)]
            )

    return gk(widx3, table_c)


def _mlp_body(x_ref, o_ref_in, wlo_ref, whi_ref, b1_ref, w2t_ref, b2_ref,
              o_ref):
    xw = x_ref[...]
    grp = jax.lax.broadcasted_iota(jnp.int32, xw.shape, 1) // 16
    xm = jnp.where(grp == o_ref_in[...], xw, 0)
    xlo = lax.bitcast_convert_type(
        lax.shift_left(xm, 16), jnp.float32
    ).astype(jnp.bfloat16)
    xhi = lax.bitcast_convert_type(
        xm & jnp.int32(-65536), jnp.float32
    ).astype(jnp.bfloat16)
    h = lax.dot_general(
        wlo_ref[...], xlo,
        dimension_numbers=(((0,), (1,)), ((), ())),
        preferred_element_type=jnp.float32,
    )
    h = h + lax.dot_general(
        whi_ref[...], xhi,
        dimension_numbers=(((0,), (1,)), ((), ())),
        preferred_element_type=jnp.float32,
    )
    h = jnp.maximum(h + b1_ref[...], 0.0).astype(jnp.bfloat16)
    z = jnp.dot(w2t_ref[...], h, preferred_element_type=jnp.float32)
    z = z + b2_ref[...]
    o_ref[...] = jax.nn.sigmoid(z)


def _mlp_tc_t(x128, o_col, Wlo, Whi, b1c, W2t, b2c):
    B = x128.shape[0]
    H = Wlo.shape[1]
    N = W2t.shape[0]
    BB = 2048
    return pl.pallas_call(
        _mlp_body,
        grid=(B // BB,),
        in_specs=[
            pl.BlockSpec((BB, 128), lambda i: (i, 0)),
            pl.BlockSpec((BB, 1), lambda i: (i, 0)),
            pl.BlockSpec((128, H), lambda i: (0, 0)),
            pl.BlockSpec((128, H), lambda i: (0, 0)),
            pl.BlockSpec((H, 1), lambda i: (0, 0)),
            pl.BlockSpec((N, H), lambda i: (0, 0)),
            pl.BlockSpec((N, 1), lambda i: (0, 0)),
        ],
        out_specs=pl.BlockSpec((N, BB), lambda i: (0, i)),
        out_shape=jax.ShapeDtypeStruct((N, B), jnp.float32),
    )(x128, o_col, Wlo, Whi, b1c, W2t, b2c)


def kernel(indices, table, W1, b1, W2, b2):
    table_t = jnp.transpose(table)
    table_c = _relayout_tc(table_t)
    p = indices.astype(jnp.int32)
    q = (p // _LB) * _LG + (p % _LG)
    widx3 = q.reshape(-1, 4, 128)
    o_col = ((p % _LB) // _LG).reshape(-1, 1)
    x128 = _gather_sc(table_c, widx3)
    Wlo = jnp.tile(W1[:16], (_G, 1)).astype(jnp.bfloat16)
    Whi = jnp.tile(W1[16:], (_G, 1)).astype(jnp.bfloat16)
    W2t = jnp.transpose(W2).astype(jnp.bfloat16)
    out_t = _mlp_tc_t(
        x128, o_col, Wlo, Whi, b1.reshape(-1, 1), W2t, b2.reshape(-1, 1)
    )
    return jnp.transpose(out_t)
